# baseline (speedup 1.0000x reference)
import math

import jax
import jax.numpy as jnp
from jax.experimental import pallas as pl

_N_NODES = 10000
_D = 128
_HEADS = 8
_N_LAYERS = 4
_N_GRAPHS = 64
_ROW_BLK = 1000


def _mm_bias_kernel(x_ref, w_ref, b_ref, o_ref):
    o_ref[...] = (
        jnp.dot(x_ref[...], w_ref[...], preferred_element_type=jnp.float32)
        + b_ref[...]
    )


def _matmul_bias(x, w, b):
    n, din = x.shape
    dout = w.shape[1]
    return pl.pallas_call(
        _mm_bias_kernel,
        grid=(n // _ROW_BLK,),
        in_specs=[
            pl.BlockSpec((_ROW_BLK, din), lambda i: (i, 0)),
            pl.BlockSpec((din, dout), lambda i: (0, 0)),
            pl.BlockSpec((1, dout), lambda i: (0, 0)),
        ],
        out_specs=pl.BlockSpec((_ROW_BLK, dout), lambda i: (i, 0)),
        out_shape=jax.ShapeDtypeStruct((n, dout), jnp.float32),
    )(x, w, b.reshape(1, dout))


def _gate_bn_kernel(out_ref, xr_ref, w1_ref, w2_ref, w3_ref, bb_ref,
                    gamma_ref, beta_ref, y_ref):
    out = out_ref[...]
    xr = xr_ref[...]
    z = jnp.sum(
        out * w1_ref[...] + xr * w2_ref[...] + (out - xr) * w3_ref[...],
        axis=1, keepdims=True,
    ) + bb_ref[0, 0]
    g = jax.nn.sigmoid(z)
    y = g * xr + (1.0 - g) * out
    mu = jnp.mean(y, axis=0, keepdims=True)
    var = jnp.mean((y - mu) ** 2, axis=0, keepdims=True)
    yn = (y - mu) / jnp.sqrt(var + 1e-5) * gamma_ref[...] + beta_ref[...]
    y_ref[...] = jnp.maximum(yn, 0.0)


def _gate_bn_relu(out, xr, wb, bb, gamma, beta):
    n, dim = out.shape
    w1 = wb[:dim].reshape(1, dim)
    w2 = wb[dim:2 * dim].reshape(1, dim)
    w3 = wb[2 * dim:].reshape(1, dim)
    return pl.pallas_call(
        _gate_bn_kernel,
        out_shape=jax.ShapeDtypeStruct((n, dim), jnp.float32),
    )(out, xr, w1, w2, w3, bb.reshape(1, 1),
      gamma.reshape(1, dim), beta.reshape(1, dim))


def _pool_kernel(h_ref, batch_ref, o_ref):
    b = batch_ref[...]
    graph_ids = jax.lax.broadcasted_iota(jnp.int32, (_N_NODES, _N_GRAPHS), 1)
    mask = (b[0][:, None] == graph_ids).astype(jnp.float32)
    sums = jax.lax.dot_general(
        mask, h_ref[...], (((0,), (0,)), ((), ())),
        preferred_element_type=jnp.float32,
    )
    counts = jnp.sum(mask, axis=0)[:, None]
    o_ref[...] = sums / jnp.maximum(counts, 1.0)


def _mean_pool(h, batch):
    return pl.pallas_call(
        _pool_kernel,
        out_shape=jax.ShapeDtypeStruct((_N_GRAPHS, h.shape[1]), jnp.float32),
    )(h, batch.reshape(1, _N_NODES))


def _edge_attention(q, k, v, src, dst, c):
    logits = jnp.sum(q[dst] * k[src], axis=-1) / math.sqrt(c)
    m = jax.ops.segment_max(logits, dst, num_segments=_N_NODES)
    m = jnp.where(jnp.isfinite(m), m, 0.0)
    e = jnp.exp(logits - m[dst])
    denom = jax.ops.segment_sum(e, dst, num_segments=_N_NODES)
    alpha = e / (denom[dst] + 1e-16)
    msg = alpha[:, :, None] * v[src]
    return jax.ops.segment_sum(msg, dst, num_segments=_N_NODES)


@jax.jit
def _forward(x, edge_index, batch, params):
    src = edge_index[0]
    dst = edge_index[1]
    h = _matmul_bias(x, params["W0"], params["b0"])
    for i, p in enumerate(params["layers"]):
        concat = i < _N_LAYERS - 1
        hc = p["Wq"].shape[1]
        c = hc // _HEADS
        wf = jnp.concatenate([p["Wq"], p["Wk"], p["Wv"], p["Ws"]], axis=1)
        bf = jnp.concatenate([p["bq"], p["bk"], p["bv"], p["bs"]])
        qkvs = _matmul_bias(h, wf, bf)
        q = qkvs[:, :hc].reshape(_N_NODES, _HEADS, c)
        k = qkvs[:, hc:2 * hc].reshape(_N_NODES, _HEADS, c)
        v = qkvs[:, 2 * hc:3 * hc].reshape(_N_NODES, _HEADS, c)
        xr = qkvs[:, 3 * hc:]
        out = _edge_attention(q, k, v, src, dst, c)
        out = out.reshape(_N_NODES, hc) if concat else jnp.mean(out, axis=1)
        h = _gate_bn_relu(out, xr, p["Wb"], p["bb"], p["gamma"], p["beta"])
    return _mean_pool(h, batch)


def kernel(x, edge_index, batch, params):
    return _forward(x, edge_index, batch, params)
